# baseline (speedup 1.0000x reference)
import functools

import jax
import jax.numpy as jnp
from jax import lax
from jax.experimental import pallas as pl
from jax.experimental.pallas import tpu as pltpu
from jax.experimental.pallas import tpu_sc as plsc

N = 10000
E = 320000
D = 128
H = 128

NC = 2
NS = 16
B_E = 80
CH = E // (NC * NS * B_E)
NPAD = 10240
RPT = NPAD // NS
LW = 16

RB = 1000
GRID_R = N // RB

CHUNK_R = 80
N_WB = RPT // CHUNK_R

_mesh = plsc.VectorSubcoreMesh(core_axis_name="c", subcore_axis_name="s")


DEG_IN_COL = 64


@functools.partial(
    pl.kernel,
    mesh=_mesh,
    out_type=jax.ShapeDtypeStruct((NC, NS, N_WB, CHUNK_R, H), jnp.float32),
    scratch_types=[
        pltpu.VMEM((B_E,), jnp.int32),
        pltpu.VMEM((B_E,), jnp.int32),
        pltpu.VMEM((B_E, H), jnp.float32),
        pltpu.VMEM((B_E, H), jnp.float32),
        pltpu.VMEM((CHUNK_R, H), jnp.float32),
        pltpu.VMEM_SHARED((NPAD, H), jnp.float32),
    ],
)
def _deg(src_hbm, dst_hbm, ones_s_hbm, ones_d_hbm, zeros_hbm, out_hbm,
         si_v, di_v, ones_s_v, ones_d_v, buf_v, deg_sh):
    c = lax.axis_index("c")
    s = lax.axis_index("s")
    pltpu.sync_copy(ones_s_hbm, ones_s_v)
    pltpu.sync_copy(ones_d_hbm, ones_d_v)
    pltpu.sync_copy(zeros_hbm, buf_v)

    def zbody(k, carry):
        pltpu.sync_copy(buf_v, deg_sh.at[pl.ds(s * RPT + k * CHUNK_R, CHUNK_R)])
        return carry

    lax.fori_loop(0, N_WB, zbody, 0)
    plsc.subcore_barrier()

    def body(j, carry):
        pltpu.sync_copy(src_hbm.at[c, s, j], si_v)
        pltpu.sync_copy(dst_hbm.at[c, s, j], di_v)
        pltpu.sync_copy(ones_s_v, deg_sh.at[si_v], add=True)
        pltpu.sync_copy(ones_d_v, deg_sh.at[di_v], add=True)
        return carry

    lax.fori_loop(0, CH, body, 0)
    plsc.subcore_barrier()

    def wbody(k, carry):
        pltpu.sync_copy(deg_sh.at[pl.ds(s * RPT + k * CHUNK_R, CHUNK_R)], buf_v)
        pltpu.sync_copy(buf_v, out_hbm.at[c, s, k])
        return carry

    lax.fori_loop(0, N_WB, wbody, 0)


@functools.partial(
    pl.kernel,
    mesh=_mesh,
    out_type=jax.ShapeDtypeStruct((NC, NS, N_WB, CHUNK_R, H), jnp.float32),
    scratch_types=[
        pltpu.VMEM((CH, B_E), jnp.int32),
        pltpu.VMEM((CH, B_E), jnp.int32),
        pltpu.VMEM((B_E, H), jnp.float32),
        pltpu.VMEM_SHARED((NPAD, H), jnp.float32),
        pltpu.SemaphoreType.DMA,
    ],
)
def _agg(xs_hbm, src_hbm, dst_hbm, zeros_hbm, out_hbm,
         src_v, dst_v, rows_v, agg_sh, sem):
    c = lax.axis_index("c")
    s = lax.axis_index("s")
    pltpu.sync_copy(src_hbm.at[c, s], src_v)
    pltpu.sync_copy(dst_hbm.at[c, s], dst_v)
    pltpu.sync_copy(zeros_hbm, rows_v)

    def zbody(k, carry):
        pltpu.sync_copy(rows_v, agg_sh.at[pl.ds(s * RPT + k * CHUNK_R, CHUNK_R)])
        return carry

    lax.fori_loop(0, N_WB, zbody, 0)
    plsc.subcore_barrier()

    def body(j, carry):
        pltpu.async_copy(xs_hbm.at[src_v.at[j]], rows_v, sem).wait()
        pltpu.sync_copy(rows_v, agg_sh.at[dst_v.at[j]], add=True)
        return carry

    lax.fori_loop(0, CH, body, 0)
    plsc.subcore_barrier()

    def wbody(k, carry):
        pltpu.sync_copy(agg_sh.at[pl.ds(s * RPT + k * CHUNK_R, CHUNK_R)], rows_v)
        pltpu.sync_copy(rows_v, out_hbm.at[c, s, k])
        return carry

    lax.fori_loop(0, N_WB, wbody, 0)


def _norms_body(deg_ref, feat_ref, wfc_ref,
                xs_ref, seq_ref, nin_ref, nout_ref):
    d_o = deg_ref[0, :, 0:1] + deg_ref[1, :, 0:1]
    d_i = (deg_ref[0, :, DEG_IN_COL:DEG_IN_COL + 1]
           + deg_ref[1, :, DEG_IN_COL:DEG_IN_COL + 1])
    n_o = jnp.where(d_o > 0, lax.rsqrt(d_o), 0.0)
    n_i = jnp.where(d_i > 0, lax.rsqrt(d_i), 0.0)
    feat = feat_ref[...]
    xs_ref[...] = feat * n_o
    seq_ref[...] = lax.dot_general(
        feat, wfc_ref[...], (((1,), (1,)), ((), ())),
        preferred_element_type=jnp.float32)
    nin_ref[...] = n_i
    nout_ref[...] = n_o


_norms = pl.pallas_call(
    _norms_body,
    grid=(GRID_R,),
    in_specs=[
        pl.BlockSpec((NC, RB, H), lambda i: (0, i, 0)),
        pl.BlockSpec((RB, D), lambda i: (i, 0)),
        pl.BlockSpec((H, D), lambda i: (0, 0)),
    ],
    out_specs=[
        pl.BlockSpec((RB, D), lambda i: (i, 0)),
        pl.BlockSpec((RB, H), lambda i: (i, 0)),
        pl.BlockSpec((RB, 1), lambda i: (i, 0)),
        pl.BlockSpec((RB, 1), lambda i: (i, 0)),
    ],
    out_shape=[
        jax.ShapeDtypeStruct((N, D), jnp.float32),
        jax.ShapeDtypeStruct((N, H), jnp.float32),
        jax.ShapeDtypeStruct((N, 1), jnp.float32),
        jax.ShapeDtypeStruct((N, 1), jnp.float32),
    ],
)


def _layer_body(agg_ref, nin_ref, nout_ref, w_ref, b_ref, xs2_ref):
    a = (agg_ref[0] + agg_ref[1]) * nin_ref[...]
    h = jnp.dot(a, w_ref[...], preferred_element_type=jnp.float32)
    h = jnp.maximum(h + b_ref[...], 0.0)
    xs2_ref[...] = h * nout_ref[...]


_layer = pl.pallas_call(
    _layer_body,
    grid=(GRID_R,),
    in_specs=[
        pl.BlockSpec((NC, RB, H), lambda i: (0, i, 0)),
        pl.BlockSpec((RB, 1), lambda i: (i, 0)),
        pl.BlockSpec((RB, 1), lambda i: (i, 0)),
        pl.BlockSpec((H, H), lambda i: (0, 0)),
        pl.BlockSpec((1, H), lambda i: (0, 0)),
    ],
    out_specs=pl.BlockSpec((RB, H), lambda i: (i, 0)),
    out_shape=jax.ShapeDtypeStruct((N, H), jnp.float32),
)


def _final_body(agg_ref, nin_ref, w_ref, b_ref, seq_ref, g_ref, be_ref,
                h_ref, hn_ref):
    a = (agg_ref[0] + agg_ref[1]) * nin_ref[...]
    h = jnp.dot(a, w_ref[...], preferred_element_type=jnp.float32)
    h = jnp.maximum(h + b_ref[...], 0.0)
    h_ref[...] = h
    h2 = h + seq_ref[...]
    mu = jnp.mean(h2, axis=-1, keepdims=True)
    var = jnp.mean((h2 - mu) ** 2, axis=-1, keepdims=True)
    hn_ref[...] = (h2 - mu) / jnp.sqrt(var + 1e-5) * g_ref[...] + be_ref[...]


_final = pl.pallas_call(
    _final_body,
    grid=(GRID_R,),
    in_specs=[
        pl.BlockSpec((NC, RB, H), lambda i: (0, i, 0)),
        pl.BlockSpec((RB, 1), lambda i: (i, 0)),
        pl.BlockSpec((H, H), lambda i: (0, 0)),
        pl.BlockSpec((1, H), lambda i: (0, 0)),
        pl.BlockSpec((RB, H), lambda i: (i, 0)),
        pl.BlockSpec((1, H), lambda i: (0, 0)),
        pl.BlockSpec((1, H), lambda i: (0, 0)),
    ],
    out_specs=[
        pl.BlockSpec((RB, H), lambda i: (i, 0)),
        pl.BlockSpec((RB, H), lambda i: (i, 0)),
    ],
    out_shape=[
        jax.ShapeDtypeStruct((N, H), jnp.float32),
        jax.ShapeDtypeStruct((N, H), jnp.float32),
    ],
)


def _adj_body(hi_ref, hj_ref, o_ref):
    hi = hi_ref[...].astype(jnp.bfloat16)
    hj = hj_ref[...].astype(jnp.bfloat16)
    logits = lax.dot_general(
        hi, hj, (((1,), (1,)), ((), ())),
        preferred_element_type=jnp.float32)
    o_ref[...] = jax.nn.sigmoid(logits)


AR = 400

_adj = pl.pallas_call(
    _adj_body,
    grid=(N // AR,),
    in_specs=[
        pl.BlockSpec((AR, H), lambda i: (i, 0)),
        pl.BlockSpec((N, H), lambda i: (0, 0)),
    ],
    out_specs=pl.BlockSpec((AR, N), lambda i: (i, 0)),
    out_shape=jax.ShapeDtypeStruct((N, N), jnp.float32),
)


def kernel(features, edge_index, W0, b0, W1, b1, Wfc, gamma, beta):
    src = edge_index[0].reshape(NC, NS, CH, B_E)
    dst = edge_index[1].reshape(NC, NS, CH, B_E)
    lane = jnp.arange(H)
    ones_s = jnp.broadcast_to((lane == 0).astype(jnp.float32), (B_E, H))
    ones_d = jnp.broadcast_to((lane == DEG_IN_COL).astype(jnp.float32), (B_E, H))
    zeros_chunk = jnp.zeros((CHUNK_R, H), jnp.float32)

    deg = _deg(src, dst, ones_s, ones_d, zeros_chunk).reshape(NC, NPAD, H)
    xs, seq_fts, n_in, n_out = _norms(deg, features, Wfc)

    p1 = _agg(xs, src, dst, zeros_chunk).reshape(NC, NPAD, H)
    xs2 = _layer(p1, n_in, n_out, W0, b0.reshape(1, H))
    p2 = _agg(xs2, src, dst, zeros_chunk).reshape(NC, NPAD, H)
    h, hn = _final(p2, n_in, W1, b1.reshape(1, H), seq_fts,
                   gamma.reshape(1, H), beta.reshape(1, H))
    adj = _adj(h, h)
    return (adj, hn, seq_fts)


# speedup vs baseline: 1.0090x; 1.0090x over previous
import functools

import jax
import jax.numpy as jnp
from jax import lax
from jax.experimental import pallas as pl
from jax.experimental.pallas import tpu as pltpu
from jax.experimental.pallas import tpu_sc as plsc

N = 10000
E = 320000
D = 128
H = 128

NC = 2
NS = 16
B_E = 80
CH = E // (NC * NS * B_E)
NPAD = 10240
RPT = NPAD // NS
LW = 16

RB = 1000
GRID_R = N // RB

CHUNK_R = 80
N_WB = RPT // CHUNK_R

_mesh = plsc.VectorSubcoreMesh(core_axis_name="c", subcore_axis_name="s")


DEG_IN_COL = 64


@functools.partial(
    pl.kernel,
    mesh=_mesh,
    out_type=jax.ShapeDtypeStruct((NC, NS, N_WB, CHUNK_R, H), jnp.float32),
    scratch_types=[
        pltpu.VMEM((B_E,), jnp.int32),
        pltpu.VMEM((B_E,), jnp.int32),
        pltpu.VMEM((B_E, H), jnp.float32),
        pltpu.VMEM((B_E, H), jnp.float32),
        pltpu.VMEM((CHUNK_R, H), jnp.float32),
        pltpu.VMEM_SHARED((NPAD, H), jnp.float32),
        pltpu.SemaphoreType.DMA,
        pltpu.SemaphoreType.DMA,
    ],
)
def _deg(src_hbm, dst_hbm, ones_s_hbm, ones_d_hbm, zeros_hbm, out_hbm,
         si_v, di_v, ones_s_v, ones_d_v, buf_v, deg_sh, dsem0, dsem1):
    c = lax.axis_index("c")
    s = lax.axis_index("s")
    pltpu.sync_copy(ones_s_hbm, ones_s_v)
    pltpu.sync_copy(ones_d_hbm, ones_d_v)
    pltpu.sync_copy(zeros_hbm, buf_v)

    def zbody(k, carry):
        pltpu.sync_copy(buf_v, deg_sh.at[pl.ds(s * RPT + k * CHUNK_R, CHUNK_R)])
        return carry

    lax.fori_loop(0, N_WB, zbody, 0)
    plsc.subcore_barrier()

    def body(j, carry):
        pltpu.sync_copy(src_hbm.at[c, s, j], si_v)
        pltpu.sync_copy(dst_hbm.at[c, s, j], di_v)
        cp_s = pltpu.make_async_copy(ones_s_v, deg_sh.at[si_v], dsem0)
        cp_d = pltpu.make_async_copy(ones_d_v, deg_sh.at[di_v], dsem1)
        cp_s.start(add=True)
        cp_d.start(add=True)
        cp_s.wait()
        cp_d.wait()
        return carry

    lax.fori_loop(0, CH, body, 0)
    plsc.subcore_barrier()

    def wbody(k, carry):
        pltpu.sync_copy(deg_sh.at[pl.ds(s * RPT + k * CHUNK_R, CHUNK_R)], buf_v)
        pltpu.sync_copy(buf_v, out_hbm.at[c, s, k])
        return carry

    lax.fori_loop(0, N_WB, wbody, 0)


@functools.partial(
    pl.kernel,
    mesh=_mesh,
    out_type=jax.ShapeDtypeStruct((NC, NS, N_WB, CHUNK_R, H), jnp.float32),
    scratch_types=[
        pltpu.VMEM((CH, B_E), jnp.int32),
        pltpu.VMEM((CH, B_E), jnp.int32),
        pltpu.VMEM((B_E, H), jnp.float32),
        pltpu.VMEM_SHARED((NPAD, H), jnp.float32),
        pltpu.SemaphoreType.DMA,
    ],
)
def _agg(xs_hbm, src_hbm, dst_hbm, zeros_hbm, out_hbm,
         src_v, dst_v, rows_v, agg_sh, sem):
    c = lax.axis_index("c")
    s = lax.axis_index("s")
    pltpu.sync_copy(src_hbm.at[c, s], src_v)
    pltpu.sync_copy(dst_hbm.at[c, s], dst_v)
    pltpu.sync_copy(zeros_hbm, rows_v)

    def zbody(k, carry):
        pltpu.sync_copy(rows_v, agg_sh.at[pl.ds(s * RPT + k * CHUNK_R, CHUNK_R)])
        return carry

    lax.fori_loop(0, N_WB, zbody, 0)
    plsc.subcore_barrier()

    def body(j, carry):
        pltpu.async_copy(xs_hbm.at[src_v.at[j]], rows_v, sem).wait()
        pltpu.sync_copy(rows_v, agg_sh.at[dst_v.at[j]], add=True)
        return carry

    lax.fori_loop(0, CH, body, 0)
    plsc.subcore_barrier()

    def wbody(k, carry):
        pltpu.sync_copy(agg_sh.at[pl.ds(s * RPT + k * CHUNK_R, CHUNK_R)], rows_v)
        pltpu.sync_copy(rows_v, out_hbm.at[c, s, k])
        return carry

    lax.fori_loop(0, N_WB, wbody, 0)


def _norms_body(deg_ref, feat_ref, wfc_ref,
                xs_ref, seq_ref, nin_ref, nout_ref):
    d_o = deg_ref[0, :, 0:1] + deg_ref[1, :, 0:1]
    d_i = (deg_ref[0, :, DEG_IN_COL:DEG_IN_COL + 1]
           + deg_ref[1, :, DEG_IN_COL:DEG_IN_COL + 1])
    n_o = jnp.where(d_o > 0, lax.rsqrt(d_o), 0.0)
    n_i = jnp.where(d_i > 0, lax.rsqrt(d_i), 0.0)
    feat = feat_ref[...]
    xs_ref[...] = feat * n_o
    seq_ref[...] = lax.dot_general(
        feat, wfc_ref[...], (((1,), (1,)), ((), ())),
        preferred_element_type=jnp.float32)
    nin_ref[...] = n_i
    nout_ref[...] = n_o


_norms = pl.pallas_call(
    _norms_body,
    grid=(GRID_R,),
    in_specs=[
        pl.BlockSpec((NC, RB, H), lambda i: (0, i, 0)),
        pl.BlockSpec((RB, D), lambda i: (i, 0)),
        pl.BlockSpec((H, D), lambda i: (0, 0)),
    ],
    out_specs=[
        pl.BlockSpec((RB, D), lambda i: (i, 0)),
        pl.BlockSpec((RB, H), lambda i: (i, 0)),
        pl.BlockSpec((RB, 1), lambda i: (i, 0)),
        pl.BlockSpec((RB, 1), lambda i: (i, 0)),
    ],
    out_shape=[
        jax.ShapeDtypeStruct((N, D), jnp.float32),
        jax.ShapeDtypeStruct((N, H), jnp.float32),
        jax.ShapeDtypeStruct((N, 1), jnp.float32),
        jax.ShapeDtypeStruct((N, 1), jnp.float32),
    ],
)


def _layer_body(agg_ref, nin_ref, nout_ref, w_ref, b_ref, xs2_ref):
    a = (agg_ref[0] + agg_ref[1]) * nin_ref[...]
    h = jnp.dot(a, w_ref[...], preferred_element_type=jnp.float32)
    h = jnp.maximum(h + b_ref[...], 0.0)
    xs2_ref[...] = h * nout_ref[...]


_layer = pl.pallas_call(
    _layer_body,
    grid=(GRID_R,),
    in_specs=[
        pl.BlockSpec((NC, RB, H), lambda i: (0, i, 0)),
        pl.BlockSpec((RB, 1), lambda i: (i, 0)),
        pl.BlockSpec((RB, 1), lambda i: (i, 0)),
        pl.BlockSpec((H, H), lambda i: (0, 0)),
        pl.BlockSpec((1, H), lambda i: (0, 0)),
    ],
    out_specs=pl.BlockSpec((RB, H), lambda i: (i, 0)),
    out_shape=jax.ShapeDtypeStruct((N, H), jnp.float32),
)


def _final_body(agg_ref, nin_ref, w_ref, b_ref, seq_ref, g_ref, be_ref,
                h_ref, hn_ref):
    a = (agg_ref[0] + agg_ref[1]) * nin_ref[...]
    h = jnp.dot(a, w_ref[...], preferred_element_type=jnp.float32)
    h = jnp.maximum(h + b_ref[...], 0.0)
    h_ref[...] = h
    h2 = h + seq_ref[...]
    mu = jnp.mean(h2, axis=-1, keepdims=True)
    var = jnp.mean((h2 - mu) ** 2, axis=-1, keepdims=True)
    hn_ref[...] = (h2 - mu) / jnp.sqrt(var + 1e-5) * g_ref[...] + be_ref[...]


_final = pl.pallas_call(
    _final_body,
    grid=(GRID_R,),
    in_specs=[
        pl.BlockSpec((NC, RB, H), lambda i: (0, i, 0)),
        pl.BlockSpec((RB, 1), lambda i: (i, 0)),
        pl.BlockSpec((H, H), lambda i: (0, 0)),
        pl.BlockSpec((1, H), lambda i: (0, 0)),
        pl.BlockSpec((RB, H), lambda i: (i, 0)),
        pl.BlockSpec((1, H), lambda i: (0, 0)),
        pl.BlockSpec((1, H), lambda i: (0, 0)),
    ],
    out_specs=[
        pl.BlockSpec((RB, H), lambda i: (i, 0)),
        pl.BlockSpec((RB, H), lambda i: (i, 0)),
    ],
    out_shape=[
        jax.ShapeDtypeStruct((N, H), jnp.float32),
        jax.ShapeDtypeStruct((N, H), jnp.float32),
    ],
)


def _adj_body(hi_ref, hj_ref, o_ref):
    logits = lax.dot_general(
        hi_ref[...], hj_ref[...], (((1,), (1,)), ((), ())),
        preferred_element_type=jnp.float32)
    o_ref[...] = jax.nn.sigmoid(logits)


AR = 400

_adj = pl.pallas_call(
    _adj_body,
    grid=(N // AR,),
    in_specs=[
        pl.BlockSpec((AR, H), lambda i: (i, 0)),
        pl.BlockSpec((N, H), lambda i: (0, 0)),
    ],
    out_specs=pl.BlockSpec((AR, N), lambda i: (i, 0)),
    out_shape=jax.ShapeDtypeStruct((N, N), jnp.float32),
)


def kernel(features, edge_index, W0, b0, W1, b1, Wfc, gamma, beta):
    src = edge_index[0].reshape(NC, NS, CH, B_E)
    dst = edge_index[1].reshape(NC, NS, CH, B_E)
    lane = jnp.arange(H)
    ones_s = jnp.broadcast_to((lane == 0).astype(jnp.float32), (B_E, H))
    ones_d = jnp.broadcast_to((lane == DEG_IN_COL).astype(jnp.float32), (B_E, H))
    zeros_chunk = jnp.zeros((CHUNK_R, H), jnp.float32)

    deg = _deg(src, dst, ones_s, ones_d, zeros_chunk).reshape(NC, NPAD, H)
    xs, seq_fts, n_in, n_out = _norms(deg, features, Wfc)

    p1 = _agg(xs, src, dst, zeros_chunk).reshape(NC, NPAD, H)
    xs2 = _layer(p1, n_in, n_out, W0, b0.reshape(1, H))
    p2 = _agg(xs2, src, dst, zeros_chunk).reshape(NC, NPAD, H)
    h, hn = _final(p2, n_in, W1, b1.reshape(1, H), seq_fts,
                   gamma.reshape(1, H), beta.reshape(1, H))
    adj = _adj(h, h)
    return (adj, hn, seq_fts)
